# Initial kernel scaffold; baseline (speedup 1.0000x reference)
import jax
import jax.numpy as jnp
from jax.experimental import pallas as pl


def kernel(x, block0_edge_index, block1_edge_index, pos_edge_index, neg_edge_index, W1, b1, W2, b2):
    raise NotImplementedError("write your pallas kernel here")


# speedup vs baseline: 1.7911x; 1.7911x over previous
import functools

import jax
import jax.numpy as jnp
from jax import lax
from jax.experimental import pallas as pl
from jax.experimental.pallas import tpu as pltpu
from jax.experimental.pallas import tpu_sc as plsc

N = 10000
E = 320000
D = 128
NC = 2
NS = 16
NW = NC * NS
L = 16
CHUNK = 128
NCHUNKS = E // CHUNK
N_PAD = 10240
ROWS_PER_TILE = N_PAD // NS

_mesh = plsc.VectorSubcoreMesh(core_axis_name="c", subcore_axis_name="s")


@functools.partial(
    pl.kernel,
    out_type=jax.ShapeDtypeStruct((NC, 4, N_PAD, L), jnp.float32),
    mesh=_mesh,
    compiler_params=pltpu.CompilerParams(
        needs_layout_passes=False, use_tc_tiling_on_sc=False),
    scratch_types=[
        pltpu.VMEM_SHARED((N_PAD, L), jnp.float32),
        pltpu.VMEM_SHARED((N_PAD, L), jnp.float32),
        pltpu.VMEM_SHARED((N_PAD, L), jnp.float32),
        pltpu.VMEM_SHARED((N_PAD, L), jnp.float32),
        pltpu.VMEM((CHUNK,), jnp.int32),
        pltpu.VMEM((CHUNK, L), jnp.float32),
        pltpu.VMEM((ROWS_PER_TILE, L), jnp.float32),
    ],
)
def _degrees_sc(ones_hbm, zz_hbm, i0, i1, i2, i3, out,
                h0, h1, h2, h3, idx_v, ones_v, zz_v):
    cid = lax.axis_index("c")
    sid = lax.axis_index("s")
    wid = sid * NC + cid
    hists = (h0, h1, h2, h3)

    pltpu.sync_copy(ones_hbm, ones_v)
    pltpu.sync_copy(zz_hbm, zz_v)

    for hist in hists:
        pltpu.sync_copy(zz_v, hist.at[pl.ds(sid * ROWS_PER_TILE, ROWS_PER_TILE), :])
    plsc.subcore_barrier()

    count = (NCHUNKS - wid + NW - 1) // NW

    for k, (src, hist) in enumerate(zip((i0, i1, i2, i3), hists)):
        def body(i, _, src=src, hist=hist):
            off = (wid + i * NW) * CHUNK
            pltpu.sync_copy(src.at[pl.ds(off, CHUNK)], idx_v)
            pltpu.sync_copy(ones_v, hist.at[idx_v], add=True)
            return 0

        lax.fori_loop(0, count, body, 0)

    plsc.subcore_barrier()
    for k, hist in enumerate(hists):
        pltpu.sync_copy(
            hist.at[pl.ds(sid * ROWS_PER_TILE, ROWS_PER_TILE), :],
            out.at[cid, k, pl.ds(sid * ROWS_PER_TILE, ROWS_PER_TILE), :],
        )


@functools.partial(
    pl.kernel,
    out_type=jax.ShapeDtypeStruct((NC, N_PAD, D), jnp.float32),
    mesh=_mesh,
    compiler_params=pltpu.CompilerParams(needs_layout_passes=False),
    scratch_types=[
        pltpu.VMEM_SHARED((N_PAD, D), jnp.float32),
        pltpu.VMEM((CHUNK,), jnp.int32),
        pltpu.VMEM((CHUNK,), jnp.int32),
        pltpu.VMEM((CHUNK, D), jnp.float32),
        pltpu.VMEM((CHUNK, D), jnp.float32),
        pltpu.SemaphoreType.DMA,
    ],
)
def _aggregate_sc(zrows_hbm, h, src, dst, out, agg, sidx_v, didx_v,
                  rows_v, zrows_v, sem):
    cid = lax.axis_index("c")
    sid = lax.axis_index("s")
    wid = sid * NC + cid

    pltpu.sync_copy(zrows_hbm, zrows_v)
    for j in range(ROWS_PER_TILE // CHUNK):
        pltpu.sync_copy(
            zrows_v, agg.at[pl.ds(sid * ROWS_PER_TILE + j * CHUNK, CHUNK), :]
        )
    plsc.subcore_barrier()

    count = (NCHUNKS - wid + NW - 1) // NW

    def body(i, _):
        off = (wid + i * NW) * CHUNK
        pltpu.sync_copy(src.at[pl.ds(off, CHUNK)], sidx_v)
        pltpu.sync_copy(dst.at[pl.ds(off, CHUNK)], didx_v)
        pltpu.async_copy(h.at[sidx_v], rows_v, sem).wait()
        pltpu.sync_copy(rows_v, agg.at[didx_v], add=True)
        return 0

    lax.fori_loop(0, count, body, 0)

    plsc.subcore_barrier()
    pltpu.sync_copy(
        agg.at[pl.ds(sid * ROWS_PER_TILE, ROWS_PER_TILE), :],
        out.at[cid, pl.ds(sid * ROWS_PER_TILE, ROWS_PER_TILE), :],
    )


@functools.partial(
    pl.kernel,
    out_type=(
        jax.ShapeDtypeStruct((E,), jnp.float32),
        jax.ShapeDtypeStruct((E,), jnp.float32),
    ),
    mesh=_mesh,
    compiler_params=pltpu.CompilerParams(needs_layout_passes=False),
    scratch_types=[
        pltpu.VMEM((CHUNK,), jnp.int32),
        pltpu.VMEM((CHUNK,), jnp.int32),
        pltpu.VMEM((CHUNK, D), jnp.float32),
        pltpu.VMEM((CHUNK, D), jnp.float32),
        pltpu.VMEM((CHUNK,), jnp.float32),
        pltpu.SemaphoreType.DMA,
    ],
)
def _edge_dot_sc(h, ps, pd, ns, nd, pos_out, neg_out,
                 sidx_v, didx_v, srows_v, drows_v, score_v, sem):
    cid = lax.axis_index("c")
    sid = lax.axis_index("s")
    wid = sid * NC + cid

    lanes = lax.iota(jnp.int32, L)

    def do_edges(sref, dref, oref):
        count = (NCHUNKS - wid + NW - 1) // NW

        def body(i, _):
            off = (wid + i * NW) * CHUNK
            pltpu.sync_copy(sref.at[pl.ds(off, CHUNK)], sidx_v)
            pltpu.sync_copy(dref.at[pl.ds(off, CHUNK)], didx_v)
            cs = pltpu.async_copy(h.at[sidx_v], srows_v, sem)
            cd = pltpu.async_copy(h.at[didx_v], drows_v, sem)
            cs.wait()
            cd.wait()
            for g in range(CHUNK // L):
                rows = lanes + g * L

                def inner(j, acc, rows=rows):
                    cols = jnp.full((L,), j, jnp.int32)
                    sv = plsc.load_gather(srows_v, [rows, cols])
                    dv = plsc.load_gather(drows_v, [rows, cols])
                    return acc + sv * dv

                acc = lax.fori_loop(
                    0, D, inner, jnp.zeros((L,), jnp.float32), unroll=4
                )
                score_v[pl.ds(g * L, L)] = acc
            pltpu.sync_copy(score_v, oref.at[pl.ds(off, CHUNK)])
            return 0

        lax.fori_loop(0, count, body, 0)

    do_edges(ps, pd, pos_out)
    do_edges(ns, nd, neg_out)


ROWS_BLK = 1000


def _norm(deg):
    return jnp.where(deg > 0, lax.rsqrt(jnp.maximum(deg, 1.0)), 0.0)


def _deg_slice(deg_ref, k):
    ds = pl.ds(pl.program_id(0) * ROWS_BLK, ROWS_BLK)
    return deg_ref[0, k, ds, 0] + deg_ref[1, k, ds, 0]


def _tc1_body(x_ref, deg_ref, w_ref, out_ref):
    deg = _deg_slice(deg_ref, 0)
    h = x_ref[...] * _norm(deg)[:, None]
    out_ref[...] = jnp.dot(h, w_ref[...], preferred_element_type=jnp.float32)


def _tc2_body(agg_ref, deg_ref, b_ref, w_ref, out_ref):
    agg = agg_ref[0] + agg_ref[1]
    nd = _norm(_deg_slice(deg_ref, 1))
    h = jax.nn.relu(agg * nd[:, None] + b_ref[...])
    ns = _norm(_deg_slice(deg_ref, 2))
    h = h * ns[:, None]
    out_ref[...] = jnp.dot(h, w_ref[...], preferred_element_type=jnp.float32)


def _tc3_body(agg_ref, deg_ref, b_ref, out_ref):
    agg = agg_ref[0] + agg_ref[1]
    nd = _norm(_deg_slice(deg_ref, 3))
    out_ref[...] = jax.nn.relu(agg * nd[:, None] + b_ref[...])


_deg_spec = pl.BlockSpec((NC, 4, N_PAD, L), lambda i: (0, 0, 0, 0))
_rows_spec = pl.BlockSpec((ROWS_BLK, D), lambda i: (i, 0))
_agg_spec = pl.BlockSpec((NC, ROWS_BLK, D), lambda i: (0, i, 0))
_w_spec = pl.BlockSpec((D, D), lambda i: (0, 0))
_b_spec = pl.BlockSpec((D,), lambda i: (0,))
_out_struct = jax.ShapeDtypeStruct((N, D), jnp.float32)

_tc1 = pl.pallas_call(
    _tc1_body, grid=(N // ROWS_BLK,),
    in_specs=[_rows_spec, _deg_spec, _w_spec],
    out_specs=_rows_spec, out_shape=_out_struct,
)
_tc2 = pl.pallas_call(
    _tc2_body, grid=(N // ROWS_BLK,),
    in_specs=[_agg_spec, _deg_spec, _b_spec, _w_spec],
    out_specs=_rows_spec, out_shape=_out_struct,
)
_tc3 = pl.pallas_call(
    _tc3_body, grid=(N // ROWS_BLK,),
    in_specs=[_agg_spec, _deg_spec, _b_spec],
    out_specs=_rows_spec, out_shape=_out_struct,
)


def kernel(x, block0_edge_index, block1_edge_index, pos_edge_index,
           neg_edge_index, W1, b1, W2, b2):
    b0s, b0d = block0_edge_index[0], block0_edge_index[1]
    b1s, b1d = block1_edge_index[0], block1_edge_index[1]

    ones_c = jnp.ones((CHUNK, L), jnp.float32)
    zz_c = jnp.zeros((ROWS_PER_TILE, L), jnp.float32)
    zrows_c = jnp.zeros((CHUNK, D), jnp.float32)

    deg = _degrees_sc(ones_c, zz_c, b0s, b0d, b1s, b1d)

    h0 = _tc1(x, deg, W1)
    agg1 = _aggregate_sc(zrows_c, h0, b0s, b0d)
    h1 = _tc2(agg1, deg, b1, W2)
    agg2 = _aggregate_sc(zrows_c, h1, b1s, b1d)
    h2 = _tc3(agg2, deg, b2)

    pos, neg = _edge_dot_sc(
        h2, pos_edge_index[0], pos_edge_index[1],
        neg_edge_index[0], neg_edge_index[1],
    )
    return (pos[:, None], neg[:, None])


# speedup vs baseline: 2.4525x; 1.3692x over previous
import functools

import jax
import jax.numpy as jnp
from jax import lax
from jax.experimental import pallas as pl
from jax.experimental.pallas import tpu as pltpu
from jax.experimental.pallas import tpu_sc as plsc

N = 10000
E = 320000
D = 128
NC = 2
NS = 16
NW = NC * NS
L = 16
CHUNK = 128
NCHT = 80
QC = 16
E_PAD = NW * NCHT * CHUNK
N_PAD = 10240
ROWS_PER_TILE = N_PAD // NS
SINK = N_PAD - 1

_mesh = plsc.VectorSubcoreMesh(core_axis_name="c", subcore_axis_name="s")


@functools.partial(
    pl.kernel,
    out_type=jax.ShapeDtypeStruct((NC, 4, N_PAD, L), jnp.float32),
    mesh=_mesh,
    compiler_params=pltpu.CompilerParams(
        needs_layout_passes=False, use_tc_tiling_on_sc=False),
    scratch_types=[
        pltpu.VMEM_SHARED((N_PAD, L), jnp.float32),
        pltpu.VMEM_SHARED((N_PAD, L), jnp.float32),
        pltpu.VMEM_SHARED((N_PAD, L), jnp.float32),
        pltpu.VMEM_SHARED((N_PAD, L), jnp.float32),
        pltpu.VMEM((NCHT, CHUNK), jnp.int32),
        pltpu.VMEM((CHUNK, L), jnp.float32),
        pltpu.VMEM((ROWS_PER_TILE, L), jnp.float32),
        pltpu.SemaphoreType.DMA,
        pltpu.SemaphoreType.DMA,
    ],
)
def _degrees_sc(ones_hbm, zz_hbm, i0, i1, i2, i3, out,
                h0, h1, h2, h3, idxs_v, ones_v, zz_v, semA, semB):
    cid = lax.axis_index("c")
    sid = lax.axis_index("s")
    wid = sid * NC + cid
    hists = (h0, h1, h2, h3)

    pltpu.sync_copy(ones_hbm, ones_v)
    pltpu.sync_copy(zz_hbm, zz_v)
    for hist in hists:
        pltpu.sync_copy(zz_v, hist.at[pl.ds(sid * ROWS_PER_TILE, ROWS_PER_TILE), :])
    plsc.subcore_barrier()

    for src, hist in zip((i0, i1, i2, i3), hists):
        pltpu.sync_copy(src.at[pl.ds(wid * NCHT, NCHT), :], idxs_v)

        def issue(c, sem, hist=hist):
            pltpu.async_copy(ones_v, hist.at[idxs_v.at[c]], sem, add=True)

        def drain(sem):
            pltpu.make_async_copy(ones_hbm, ones_v, sem).wait()

        issue(0, semA)
        issue(1, semB)

        def body(j, _, issue=issue, drain=drain):
            drain(semA)
            issue(2 * j + 2, semA)
            drain(semB)
            issue(2 * j + 3, semB)
            return 0

        lax.fori_loop(0, (NCHT - 2) // 2, body, 0)
        drain(semA)
        drain(semB)

    plsc.subcore_barrier()
    for k, hist in enumerate(hists):
        pltpu.sync_copy(
            hist.at[pl.ds(sid * ROWS_PER_TILE, ROWS_PER_TILE), :],
            out.at[cid, k, pl.ds(sid * ROWS_PER_TILE, ROWS_PER_TILE), :],
        )


@functools.partial(
    pl.kernel,
    out_type=jax.ShapeDtypeStruct((NC, N_PAD, D), jnp.float32),
    mesh=_mesh,
    compiler_params=pltpu.CompilerParams(needs_layout_passes=False),
    scratch_types=[
        pltpu.VMEM_SHARED((N_PAD, D), jnp.float32),
        pltpu.VMEM((QC, CHUNK), jnp.int32),
        pltpu.VMEM((QC, CHUNK), jnp.int32),
        pltpu.VMEM((CHUNK, D), jnp.float32),
        pltpu.VMEM((CHUNK, D), jnp.float32),
        pltpu.SemaphoreType.DMA,
        pltpu.SemaphoreType.DMA,
        pltpu.SemaphoreType.DMA,
        pltpu.SemaphoreType.DMA,
    ],
)
def _aggregate_sc(zrows_hbm, h, src, dst, out, agg, sidx_v, didx_v,
                  rows0, rows1, g0, g1, s0, s1):
    cid = lax.axis_index("c")
    sid = lax.axis_index("s")
    wid = sid * NC + cid

    for j in range(ROWS_PER_TILE // CHUNK):
        pltpu.sync_copy(
            zrows_hbm, agg.at[pl.ds(sid * ROWS_PER_TILE + j * CHUNK, CHUNK), :]
        )
    plsc.subcore_barrier()

    def gather(c, buf, sem):
        pltpu.async_copy(h.at[sidx_v.at[c]], buf, sem)

    def drain_g(buf, sem):
        pltpu.make_async_copy(h.at[pl.ds(0, CHUNK), :], buf, sem).wait()

    def scat(c, buf, sem):
        pltpu.async_copy(buf, agg.at[didx_v.at[c]], sem, add=True)

    def drain_s(buf, sem):
        pltpu.make_async_copy(zrows_hbm, buf, sem).wait()

    for q in range(NCHT // QC):
        qbase = wid * NCHT + q * QC
        pltpu.sync_copy(src.at[pl.ds(qbase, QC), :], sidx_v)
        pltpu.sync_copy(dst.at[pl.ds(qbase, QC), :], didx_v)

        gather(0, rows0, g0)
        gather(1, rows1, g1)

        def body(j, _):
            drain_g(rows0, g0)
            scat(2 * j, rows0, s0)
            drain_g(rows1, g1)
            scat(2 * j + 1, rows1, s1)
            drain_s(rows0, s0)
            gather(2 * j + 2, rows0, g0)
            drain_s(rows1, s1)
            gather(2 * j + 3, rows1, g1)
            return 0

        lax.fori_loop(0, (QC - 2) // 2, body, 0)
        drain_g(rows0, g0)
        scat(QC - 2, rows0, s0)
        drain_g(rows1, g1)
        scat(QC - 1, rows1, s1)
        drain_s(rows0, s0)
        drain_s(rows1, s1)

    plsc.subcore_barrier()
    pltpu.sync_copy(
        agg.at[pl.ds(sid * ROWS_PER_TILE, ROWS_PER_TILE), :],
        out.at[cid, pl.ds(sid * ROWS_PER_TILE, ROWS_PER_TILE), :],
    )


@functools.partial(
    pl.kernel,
    out_type=(
        jax.ShapeDtypeStruct((E_PAD,), jnp.float32),
        jax.ShapeDtypeStruct((E_PAD,), jnp.float32),
    ),
    mesh=_mesh,
    compiler_params=pltpu.CompilerParams(needs_layout_passes=False),
    scratch_types=[
        pltpu.VMEM((NCHT, CHUNK), jnp.int32),
        pltpu.VMEM((NCHT, CHUNK), jnp.int32),
        pltpu.VMEM((CHUNK, D), jnp.float32),
        pltpu.VMEM((CHUNK, D), jnp.float32),
        pltpu.VMEM((CHUNK, D), jnp.float32),
        pltpu.VMEM((CHUNK, D), jnp.float32),
        pltpu.VMEM((NCHT * CHUNK,), jnp.float32),
        pltpu.SemaphoreType.DMA,
        pltpu.SemaphoreType.DMA,
    ],
)
def _edge_dot_sc(h, ps, pd, ns, nd, pos_out, neg_out,
                 sidx_v, didx_v, srows0, drows0, srows1, drows1,
                 score_v, g0, g1):
    cid = lax.axis_index("c")
    sid = lax.axis_index("s")
    wid = sid * NC + cid
    lanes = lax.iota(jnp.int32, L)

    def gpair(c, sb, db, sem):
        pltpu.async_copy(h.at[sidx_v.at[c]], sb, sem)
        pltpu.async_copy(h.at[didx_v.at[c]], db, sem)

    def drain2(sb, db, sem):
        pltpu.make_async_copy(h.at[pl.ds(0, CHUNK), :], sb, sem).wait()
        pltpu.make_async_copy(h.at[pl.ds(0, CHUNK), :], db, sem).wait()

    def chunk_compute(c, sb, db):
        base = c * CHUNK
        for g in range(CHUNK // L):
            def edge(i, scores, g=g):
                e = g * L + i
                acc = sb[e, pl.ds(0, L)] * db[e, pl.ds(0, L)]
                for k in range(1, D // L):
                    acc = acc + sb[e, pl.ds(k * L, L)] * db[e, pl.ds(k * L, L)]
                tot = jnp.sum(acc)
                return jnp.where(lanes == i, tot, scores)

            scores = lax.fori_loop(0, L, edge, jnp.zeros((L,), jnp.float32))
            score_v[pl.ds(base + g * L, L)] = scores

    for sref, dref, oref in ((ps, pd, pos_out), (ns, nd, neg_out)):
        pltpu.sync_copy(sref.at[pl.ds(wid * NCHT, NCHT), :], sidx_v)
        pltpu.sync_copy(dref.at[pl.ds(wid * NCHT, NCHT), :], didx_v)
        gpair(0, srows0, drows0, g0)
        gpair(1, srows1, drows1, g1)

        def body(j, _):
            drain2(srows0, drows0, g0)
            chunk_compute(2 * j, srows0, drows0)
            gpair(2 * j + 2, srows0, drows0, g0)
            drain2(srows1, drows1, g1)
            chunk_compute(2 * j + 1, srows1, drows1)
            gpair(2 * j + 3, srows1, drows1, g1)
            return 0

        lax.fori_loop(0, (NCHT - 2) // 2, body, 0)
        drain2(srows0, drows0, g0)
        chunk_compute(NCHT - 2, srows0, drows0)
        drain2(srows1, drows1, g1)
        chunk_compute(NCHT - 1, srows1, drows1)
        pltpu.sync_copy(score_v, oref.at[pl.ds(wid * NCHT * CHUNK, NCHT * CHUNK)])


ROWS_BLK = 1000


def _norm(deg):
    return jnp.where(deg > 0, lax.rsqrt(jnp.maximum(deg, 1.0)), 0.0)


def _deg_slice(deg_ref, k):
    ds = pl.ds(pl.program_id(0) * ROWS_BLK, ROWS_BLK)
    return deg_ref[0, k, ds, 0] + deg_ref[1, k, ds, 0]


def _tc1_body(x_ref, deg_ref, w_ref, out_ref):
    deg = _deg_slice(deg_ref, 0)
    h = x_ref[...] * _norm(deg)[:, None]
    out_ref[...] = jnp.dot(h, w_ref[...], preferred_element_type=jnp.float32)


def _tc2_body(agg_ref, deg_ref, b_ref, w_ref, out_ref):
    agg = agg_ref[0] + agg_ref[1]
    nd = _norm(_deg_slice(deg_ref, 1))
    h = jax.nn.relu(agg * nd[:, None] + b_ref[...])
    ns = _norm(_deg_slice(deg_ref, 2))
    h = h * ns[:, None]
    out_ref[...] = jnp.dot(h, w_ref[...], preferred_element_type=jnp.float32)


def _tc3_body(agg_ref, deg_ref, b_ref, out_ref):
    agg = agg_ref[0] + agg_ref[1]
    nd = _norm(_deg_slice(deg_ref, 3))
    out_ref[...] = jax.nn.relu(agg * nd[:, None] + b_ref[...])


_deg_spec = pl.BlockSpec((NC, 4, N_PAD, L), lambda i: (0, 0, 0, 0))
_rows_spec = pl.BlockSpec((ROWS_BLK, D), lambda i: (i, 0))
_agg_spec = pl.BlockSpec((NC, ROWS_BLK, D), lambda i: (0, i, 0))
_w_spec = pl.BlockSpec((D, D), lambda i: (0, 0))
_b_spec = pl.BlockSpec((D,), lambda i: (0,))
_out_struct = jax.ShapeDtypeStruct((N_PAD, D), jnp.float32)

_tc1 = pl.pallas_call(
    _tc1_body, grid=(N // ROWS_BLK,),
    in_specs=[_rows_spec, _deg_spec, _w_spec],
    out_specs=_rows_spec, out_shape=_out_struct,
)
_tc2 = pl.pallas_call(
    _tc2_body, grid=(N // ROWS_BLK,),
    in_specs=[_agg_spec, _deg_spec, _b_spec, _w_spec],
    out_specs=_rows_spec, out_shape=_out_struct,
)
_tc3 = pl.pallas_call(
    _tc3_body, grid=(N // ROWS_BLK,),
    in_specs=[_agg_spec, _deg_spec, _b_spec],
    out_specs=_rows_spec, out_shape=_out_struct,
)


def _chunked(idx, pad_val):
    pad = jnp.full((E_PAD - E,), pad_val, idx.dtype)
    return jnp.concatenate([idx, pad]).reshape(NW * NCHT, CHUNK)


def kernel(x, block0_edge_index, block1_edge_index, pos_edge_index,
           neg_edge_index, W1, b1, W2, b2):
    b0s = _chunked(block0_edge_index[0], SINK)
    b0d = _chunked(block0_edge_index[1], SINK)
    b1s = _chunked(block1_edge_index[0], SINK)
    b1d = _chunked(block1_edge_index[1], SINK)
    pes = _chunked(pos_edge_index[0], 0)
    ped = _chunked(pos_edge_index[1], 0)
    nes = _chunked(neg_edge_index[0], 0)
    ned = _chunked(neg_edge_index[1], 0)

    ones_c = jnp.ones((CHUNK, L), jnp.float32)
    zz_c = jnp.zeros((ROWS_PER_TILE, L), jnp.float32)
    zrows_c = jnp.zeros((CHUNK, D), jnp.float32)

    deg = _degrees_sc(ones_c, zz_c, b0s, b0d, b1s, b1d)

    h0 = _tc1(x, deg, W1)
    agg1 = _aggregate_sc(zrows_c, h0, b0s, b0d)
    h1 = _tc2(agg1, deg, b1, W2)
    agg2 = _aggregate_sc(zrows_c, h1, b1s, b1d)
    h2 = _tc3(agg2, deg, b2)

    pos, neg = _edge_dot_sc(h2, pes, ped, nes, ned)
    return (pos[:E, None], neg[:E, None])


# speedup vs baseline: 2.5912x; 1.0566x over previous
import functools

import jax
import jax.numpy as jnp
from jax import lax
from jax.experimental import pallas as pl
from jax.experimental.pallas import tpu as pltpu
from jax.experimental.pallas import tpu_sc as plsc

N = 10000
E = 320000
D = 128
NC = 2
NS = 16
NW = NC * NS
L = 16
CHUNK = 128
NCHT = 80
QC = 16
E_PAD = NW * NCHT * CHUNK
N_PAD = 10240
ROWS_PER_TILE = N_PAD // NS
SINK = N_PAD - 1

_mesh = plsc.VectorSubcoreMesh(core_axis_name="c", subcore_axis_name="s")


@functools.partial(
    pl.kernel,
    out_type=jax.ShapeDtypeStruct((NC, 4, N_PAD, L), jnp.float32),
    mesh=_mesh,
    compiler_params=pltpu.CompilerParams(
        needs_layout_passes=False, use_tc_tiling_on_sc=False),
    scratch_types=[
        pltpu.VMEM_SHARED((N_PAD, L), jnp.float32),
        pltpu.VMEM_SHARED((N_PAD, L), jnp.float32),
        pltpu.VMEM_SHARED((N_PAD, L), jnp.float32),
        pltpu.VMEM_SHARED((N_PAD, L), jnp.float32),
        pltpu.VMEM((NCHT, CHUNK), jnp.int32),
        pltpu.VMEM((CHUNK, L), jnp.float32),
        pltpu.VMEM((ROWS_PER_TILE, L), jnp.float32),
        pltpu.SemaphoreType.DMA,
        pltpu.SemaphoreType.DMA,
    ],
)
def _degrees_sc(ones_hbm, zz_hbm, i0, i1, i2, i3, out,
                h0, h1, h2, h3, idxs_v, ones_v, zz_v, semA, semB):
    cid = lax.axis_index("c")
    sid = lax.axis_index("s")
    wid = sid * NC + (1 - cid)
    hists = (h0, h1, h2, h3)

    pltpu.sync_copy(ones_hbm, ones_v)
    pltpu.sync_copy(zz_hbm, zz_v)
    for hist in hists:
        pltpu.sync_copy(zz_v, hist.at[pl.ds(sid * ROWS_PER_TILE, ROWS_PER_TILE), :])
    plsc.subcore_barrier()

    for src, hist in zip((i0, i1, i2, i3), hists):
        pltpu.sync_copy(src.at[pl.ds(wid * NCHT, NCHT), :], idxs_v)

        def issue(c, sem, hist=hist):
            pltpu.async_copy(ones_v, hist.at[idxs_v.at[c]], sem, add=True)

        def drain(sem):
            pltpu.make_async_copy(ones_hbm, ones_v, sem).wait()

        issue(0, semA)
        issue(1, semB)

        def body(j, _, issue=issue, drain=drain):
            drain(semA)
            issue(2 * j + 2, semA)
            drain(semB)
            issue(2 * j + 3, semB)
            return 0

        lax.fori_loop(0, (NCHT - 2) // 2, body, 0)
        drain(semA)
        drain(semB)

    plsc.subcore_barrier()
    for k, hist in enumerate(hists):
        pltpu.sync_copy(
            hist.at[pl.ds(sid * ROWS_PER_TILE, ROWS_PER_TILE), :],
            out.at[cid, k, pl.ds(sid * ROWS_PER_TILE, ROWS_PER_TILE), :],
        )


@functools.partial(
    pl.kernel,
    out_type=jax.ShapeDtypeStruct((NC, N_PAD, D), jnp.float32),
    mesh=_mesh,
    compiler_params=pltpu.CompilerParams(needs_layout_passes=False),
    scratch_types=[
        pltpu.VMEM_SHARED((N_PAD, D), jnp.float32),
        pltpu.VMEM((QC, CHUNK), jnp.int32),
        pltpu.VMEM((QC, CHUNK), jnp.int32),
        pltpu.VMEM((CHUNK, D), jnp.float32),
        pltpu.VMEM((CHUNK, D), jnp.float32),
        pltpu.SemaphoreType.DMA,
        pltpu.SemaphoreType.DMA,
        pltpu.SemaphoreType.DMA,
        pltpu.SemaphoreType.DMA,
    ],
)
def _aggregate_sc(zrows_hbm, h, src, dst, out, agg, sidx_v, didx_v,
                  rows0, rows1, g0, g1, s0, s1):
    cid = lax.axis_index("c")
    sid = lax.axis_index("s")
    wid = sid * NC + (1 - cid)

    for j in range(ROWS_PER_TILE // CHUNK):
        pltpu.sync_copy(
            zrows_hbm, agg.at[pl.ds(sid * ROWS_PER_TILE + j * CHUNK, CHUNK), :]
        )
    plsc.subcore_barrier()

    def gather(c, buf, sem):
        pltpu.async_copy(h.at[sidx_v.at[c]], buf, sem)

    def drain_g(buf, sem):
        pltpu.make_async_copy(h.at[pl.ds(0, CHUNK), :], buf, sem).wait()

    def scat(c, buf, sem):
        pltpu.async_copy(buf, agg.at[didx_v.at[c]], sem, add=True)

    def drain_s(buf, sem):
        pltpu.make_async_copy(zrows_hbm, buf, sem).wait()

    for q in range(NCHT // QC):
        qbase = wid * NCHT + q * QC
        pltpu.sync_copy(src.at[pl.ds(qbase, QC), :], sidx_v)
        pltpu.sync_copy(dst.at[pl.ds(qbase, QC), :], didx_v)

        gather(0, rows0, g0)
        gather(1, rows1, g1)

        def body(j, _):
            drain_g(rows0, g0)
            scat(2 * j, rows0, s0)
            drain_g(rows1, g1)
            scat(2 * j + 1, rows1, s1)
            drain_s(rows0, s0)
            gather(2 * j + 2, rows0, g0)
            drain_s(rows1, s1)
            gather(2 * j + 3, rows1, g1)
            return 0

        lax.fori_loop(0, (QC - 2) // 2, body, 0)
        drain_g(rows0, g0)
        scat(QC - 2, rows0, s0)
        drain_g(rows1, g1)
        scat(QC - 1, rows1, s1)
        drain_s(rows0, s0)
        drain_s(rows1, s1)

    plsc.subcore_barrier()
    pltpu.sync_copy(
        agg.at[pl.ds(sid * ROWS_PER_TILE, ROWS_PER_TILE), :],
        out.at[cid, pl.ds(sid * ROWS_PER_TILE, ROWS_PER_TILE), :],
    )


@functools.partial(
    pl.kernel,
    out_type=(
        jax.ShapeDtypeStruct((E_PAD,), jnp.float32),
        jax.ShapeDtypeStruct((E_PAD,), jnp.float32),
    ),
    mesh=_mesh,
    compiler_params=pltpu.CompilerParams(needs_layout_passes=False),
    scratch_types=[
        pltpu.VMEM((NCHT, CHUNK), jnp.int32),
        pltpu.VMEM((NCHT, CHUNK), jnp.int32),
        pltpu.VMEM((CHUNK, D), jnp.float32),
        pltpu.VMEM((CHUNK, D), jnp.float32),
        pltpu.VMEM((CHUNK, D), jnp.float32),
        pltpu.VMEM((CHUNK, D), jnp.float32),
        pltpu.VMEM((NCHT * CHUNK,), jnp.float32),
        pltpu.SemaphoreType.DMA,
        pltpu.SemaphoreType.DMA,
    ],
)
def _edge_dot_sc(h, ps, pd, ns, nd, pos_out, neg_out,
                 sidx_v, didx_v, srows0, drows0, srows1, drows1,
                 score_v, g0, g1):
    cid = lax.axis_index("c")
    sid = lax.axis_index("s")
    wid = sid * NC + (1 - cid)
    lanes = lax.iota(jnp.int32, L)

    def gpair(c, sb, db, sem):
        pltpu.async_copy(h.at[sidx_v.at[c]], sb, sem)
        pltpu.async_copy(h.at[didx_v.at[c]], db, sem)

    def drain2(sb, db, sem):
        pltpu.make_async_copy(h.at[pl.ds(0, CHUNK), :], sb, sem).wait()
        pltpu.make_async_copy(h.at[pl.ds(0, CHUNK), :], db, sem).wait()

    def chunk_compute(c, sb, db):
        base = c * CHUNK
        for g in range(CHUNK // L):
            def edge(i, scores, g=g):
                e = g * L + i
                acc = sb[e, pl.ds(0, L)] * db[e, pl.ds(0, L)]
                for k in range(1, D // L):
                    acc = acc + sb[e, pl.ds(k * L, L)] * db[e, pl.ds(k * L, L)]
                tot = jnp.sum(acc)
                return jnp.where(lanes == i, tot, scores)

            scores = lax.fori_loop(0, L, edge, jnp.zeros((L,), jnp.float32))
            score_v[pl.ds(base + g * L, L)] = scores

    for sref, dref, oref in ((ps, pd, pos_out), (ns, nd, neg_out)):
        pltpu.sync_copy(sref.at[pl.ds(wid * NCHT, NCHT), :], sidx_v)
        pltpu.sync_copy(dref.at[pl.ds(wid * NCHT, NCHT), :], didx_v)
        gpair(0, srows0, drows0, g0)
        gpair(1, srows1, drows1, g1)

        def body(j, _):
            drain2(srows0, drows0, g0)
            chunk_compute(2 * j, srows0, drows0)
            gpair(2 * j + 2, srows0, drows0, g0)
            drain2(srows1, drows1, g1)
            chunk_compute(2 * j + 1, srows1, drows1)
            gpair(2 * j + 3, srows1, drows1, g1)
            return 0

        lax.fori_loop(0, (NCHT - 2) // 2, body, 0)
        drain2(srows0, drows0, g0)
        chunk_compute(NCHT - 2, srows0, drows0)
        drain2(srows1, drows1, g1)
        chunk_compute(NCHT - 1, srows1, drows1)
        pltpu.sync_copy(score_v, oref.at[pl.ds(wid * NCHT * CHUNK, NCHT * CHUNK)])


ROWS_BLK = 1000


def _norm(deg):
    return jnp.where(deg > 0, lax.rsqrt(jnp.maximum(deg, 1.0)), 0.0)


def _deg_slice(deg_ref, k):
    ds = pl.ds(pl.program_id(0) * ROWS_BLK, ROWS_BLK)
    return deg_ref[0, k, ds, 0] + deg_ref[1, k, ds, 0]


def _tc1_body(x_ref, deg_ref, w_ref, out_ref):
    deg = _deg_slice(deg_ref, 0)
    h = x_ref[...] * _norm(deg)[:, None]
    out_ref[...] = jnp.dot(h, w_ref[...], preferred_element_type=jnp.float32)


def _tc2_body(agg_ref, deg_ref, b_ref, w_ref, out_ref):
    agg = agg_ref[0] + agg_ref[1]
    nd = _norm(_deg_slice(deg_ref, 1))
    h = jax.nn.relu(agg * nd[:, None] + b_ref[...])
    ns = _norm(_deg_slice(deg_ref, 2))
    h = h * ns[:, None]
    out_ref[...] = jnp.dot(h, w_ref[...], preferred_element_type=jnp.float32)


def _tc3_body(agg_ref, deg_ref, b_ref, out_ref):
    agg = agg_ref[0] + agg_ref[1]
    nd = _norm(_deg_slice(deg_ref, 3))
    out_ref[...] = jax.nn.relu(agg * nd[:, None] + b_ref[...])


_deg_spec = pl.BlockSpec((NC, 4, N_PAD, L), lambda i: (0, 0, 0, 0))
_rows_spec = pl.BlockSpec((ROWS_BLK, D), lambda i: (i, 0))
_agg_spec = pl.BlockSpec((NC, ROWS_BLK, D), lambda i: (0, i, 0))
_w_spec = pl.BlockSpec((D, D), lambda i: (0, 0))
_b_spec = pl.BlockSpec((D,), lambda i: (0,))
_out_struct = jax.ShapeDtypeStruct((N_PAD, D), jnp.float32)

_tc1 = pl.pallas_call(
    _tc1_body, grid=(N // ROWS_BLK,),
    in_specs=[_rows_spec, _deg_spec, _w_spec],
    out_specs=_rows_spec, out_shape=_out_struct,
)
_tc2 = pl.pallas_call(
    _tc2_body, grid=(N // ROWS_BLK,),
    in_specs=[_agg_spec, _deg_spec, _b_spec, _w_spec],
    out_specs=_rows_spec, out_shape=_out_struct,
)
_tc3 = pl.pallas_call(
    _tc3_body, grid=(N // ROWS_BLK,),
    in_specs=[_agg_spec, _deg_spec, _b_spec],
    out_specs=_rows_spec, out_shape=_out_struct,
)


def _chunked(idx, pad_val):
    pad = jnp.full((E_PAD - E,), pad_val, idx.dtype)
    return jnp.concatenate([idx, pad]).reshape(NW * NCHT, CHUNK)


def kernel(x, block0_edge_index, block1_edge_index, pos_edge_index,
           neg_edge_index, W1, b1, W2, b2):
    b0s = _chunked(block0_edge_index[0], SINK)
    b0d = _chunked(block0_edge_index[1], SINK)
    b1s = _chunked(block1_edge_index[0], SINK)
    b1d = _chunked(block1_edge_index[1], SINK)
    pes = _chunked(pos_edge_index[0], 0)
    ped = _chunked(pos_edge_index[1], 0)
    nes = _chunked(neg_edge_index[0], 0)
    ned = _chunked(neg_edge_index[1], 0)

    ones_c = jnp.ones((CHUNK, L), jnp.float32)
    zz_c = jnp.zeros((ROWS_PER_TILE, L), jnp.float32)
    zrows_c = jnp.zeros((CHUNK, D), jnp.float32)

    deg = _degrees_sc(ones_c, zz_c, b0s, b0d, b1s, b1d)

    h0 = _tc1(x, deg, W1)
    agg1 = _aggregate_sc(zrows_c, h0, b0s, b0d)
    h1 = _tc2(agg1, deg, b1, W2)
    agg2 = _aggregate_sc(zrows_c, h1, b1s, b1d)
    h2 = _tc3(agg2, deg, b2)

    pos, neg = _edge_dot_sc(h2, pes, ped, nes, ned)
    return (pos[:E, None], neg[:E, None])
